# baseline (speedup 1.0000x reference)
import jax
import jax.numpy as jnp
from jax.experimental import pallas as pl
from jax.experimental.pallas import tpu as pltpu

_N, _K, _D = 4096, 512, 64
_TN = 2048


def _assign_kernel(xt_ref, c_ref, out_ref, cn_ref, c4_ref):
    @pl.when(pl.program_id(0) == 0)
    def _():
        c = c_ref[...]
        cn_ref[...] = 0.5 * jnp.sum(c * c, axis=1)[:, None]
        ch = c.astype(jnp.bfloat16)
        cl = (c - ch.astype(jnp.float32)).astype(jnp.bfloat16)
        c4_ref[...] = jnp.concatenate([ch, cl, ch, cl], axis=1)

    xt = xt_ref[...]
    xh = xt.astype(jnp.bfloat16)
    xl = (xt - xh.astype(jnp.float32)).astype(jnp.bfloat16)
    x4 = jnp.concatenate([xh, xh, xl, xl], axis=0)
    scores = jax.lax.dot_general(
        c4_ref[...], x4,
        dimension_numbers=(((1,), (0,)), ((), ())),
        preferred_element_type=jnp.float32,
    )
    dist = cn_ref[...] - scores
    out_ref[...] = jnp.argmin(dist, axis=0).astype(jnp.int32)


def kernel(x, centers):
    xt = x.T
    return pl.pallas_call(
        _assign_kernel,
        grid=(_N // _TN,),
        in_specs=[
            pl.BlockSpec((_D, _TN), lambda i: (0, i)),
            pl.BlockSpec((_K, _D), lambda i: (0, 0)),
        ],
        out_specs=pl.BlockSpec((_TN,), lambda i: (i,)),
        out_shape=jax.ShapeDtypeStruct((_N,), jnp.int32),
        scratch_shapes=[
            pltpu.VMEM((_K, 1), jnp.float32),
            pltpu.VMEM((_K, 4 * _D), jnp.bfloat16),
        ],
    )(xt, centers)
